# baseline (speedup 1.0000x reference)
import functools

import jax
import jax.numpy as jnp
from jax import lax
from jax.experimental import pallas as pl
from jax.experimental.pallas import tpu as pltpu
from jax.experimental.pallas import tpu_sc as plsc

D_MODEL = 1024
NC = 2
NS = 16
NW = NC * NS
CH = 16
NBUF = 5


def _emb_body(batch, seq, ids_hbm, am_hbm, wte_hbm, wpe_hbm,
              out_hbm, mask_hbm, idx_v, b0, b1, b2, b3, b4, wq0, wq1, am_v,
              g0, g1, g2, g3, g4, o0, o1, o2, o3, o4, w0, w1, m0):
    bufs = (b0, b1, b2, b3, b4)
    gsem = (g0, g1, g2, g3, g4)
    osem = (o0, o1, o2, o3, o4)
    wq = (wq0, wq1)
    wsem = (w0, w1)
    msem = m0
    wid = lax.axis_index("s") * NC + lax.axis_index("c")
    pos_w = seq // NW
    pos_base = wid * pos_w
    nq = pos_w // CH
    n_chunks = batch * nq

    def load_wq(q):
        pltpu.async_copy(wpe_hbm.at[pl.ds(pos_base + q * CH, CH)],
                         wq[q % 2], wsem[q % 2])

    load_wq(0)
    load_wq(1)
    wpb = seq // 256
    mb = wid // wpb
    mcol = (wid % wpb) * 256
    idx_cps = []
    for b in range(batch):
        idx_cps.append(pltpu.async_copy(
            ids_hbm.at[b, pl.ds(pos_base, pos_w)],
            idx_v.at[pl.ds(b * pos_w, pos_w)], osem[b]))
    am_cp = pltpu.async_copy(am_hbm.at[mb, pl.ds(mcol, 256)], am_v, msem)
    for cp in idx_cps:
        cp.wait()

    g_cps = {}
    out_cps = {}

    def gather(i):
        q, b = i // batch, i % batch
        s = i % NBUF
        g_cps[i] = pltpu.async_copy(
            wte_hbm.at[idx_v.at[pl.ds(b * pos_w + q * CH, CH)]],
            bufs[s], gsem[s])

    for i in range(min(NBUF, n_chunks)):
        gather(i)

    am_cp.wait()
    for i in range(256 // 16):
        v = am_v[pl.ds(i * 16, 16)]
        am_v[pl.ds(i * 16, 16)] = (1.0 - v) * (-10000.0)
    am_st = pltpu.async_copy(am_v, mask_hbm.at[0, 0, mb, pl.ds(mcol, 256)],
                             msem)

    for g in range(nq):
        i0 = g * batch
        pltpu.make_async_copy(
            wpe_hbm.at[pl.ds(pos_base + g * CH, CH)],
            wq[g % 2], wsem[g % 2]).wait()
        for b in range(batch):
            g_cps[i0 + b].wait()

        group = tuple(bufs[(i0 + b) % NBUF] for b in range(batch))

        def add_row(r, _, bs=group, w=wq[g % 2]):
            for k in range(D_MODEL // 16):
                sl = pl.ds(k * 16, 16)
                for bf in bs:
                    plsc.addupdate(bf.at[r, sl], w[r, sl])
            return 0
        lax.fori_loop(0, CH, add_row, 0)
        if g + 2 < nq:
            load_wq(g + 2)

        row0 = pos_base + g * CH
        for b in range(batch):
            j = i0 + b
            out_cps[j] = pltpu.async_copy(
                bufs[j % NBUF], out_hbm.at[b, pl.ds(row0, CH)], osem[j % NBUF])
        for j in range(i0 + NBUF, min(i0 + NBUF + batch, n_chunks)):
            out_cps[j - NBUF].wait()
            gather(j)

    for j in range(max(0, n_chunks - NBUF), n_chunks):
        out_cps[j].wait()
    am_st.wait()


@functools.lru_cache(maxsize=None)
def _build(batch, seq):
    mesh = plsc.VectorSubcoreMesh(core_axis_name="c", subcore_axis_name="s")
    pos_w = seq // NW
    return pl.kernel(
        functools.partial(_emb_body, batch, seq),
        out_type=(
            jax.ShapeDtypeStruct((batch, seq, D_MODEL), jnp.float32),
            jax.ShapeDtypeStruct((1, 1, batch, seq), jnp.float32),
        ),
        mesh=mesh,
        scratch_types=[
            pltpu.VMEM((batch * pos_w,), jnp.int32),
            pltpu.VMEM((CH, D_MODEL), jnp.float32),
            pltpu.VMEM((CH, D_MODEL), jnp.float32),
            pltpu.VMEM((CH, D_MODEL), jnp.float32),
            pltpu.VMEM((CH, D_MODEL), jnp.float32),
            pltpu.VMEM((CH, D_MODEL), jnp.float32),
            pltpu.VMEM((CH, D_MODEL), jnp.float32),
            pltpu.VMEM((CH, D_MODEL), jnp.float32),
            pltpu.VMEM((256,), jnp.float32),
        ] + [pltpu.SemaphoreType.DMA] * 13,
    )


def kernel(input_ids, attention_mask, wte, wpe):
    batch, seq = input_ids.shape
    ids = input_ids if input_ids.dtype == jnp.int32 else input_ids.astype(jnp.int32)
    am = (attention_mask if attention_mask.dtype == jnp.float32
          else attention_mask.astype(jnp.float32))
    hidden, ext_mask = _build(batch, seq)(ids, am, wte, wpe)
    return (hidden, ext_mask)


# speedup vs baseline: 1.1230x; 1.1230x over previous
import functools

import jax
import jax.numpy as jnp
from jax import lax
from jax.experimental import pallas as pl
from jax.experimental.pallas import tpu as pltpu
from jax.experimental.pallas import tpu_sc as plsc

D_MODEL = 1024
NC = 2
NS = 16
NW = NC * NS
CH = 16
NBUF = 5


def _emb_body(batch, seq, ids_hbm, am_hbm, wte_hbm, wpe_hbm,
              out_hbm, mask_hbm, idx_v, b0, b1, b2, b3, b4, wq0, wq1, am_v,
              g0, g1, g2, g3, g4, o0, o1, o2, o3, o4, w0, w1, m0):
    bufs = (b0, b1, b2, b3, b4)
    gsem = (g0, g1, g2, g3, g4)
    osem = (o0, o1, o2, o3, o4)
    wq = (wq0, wq1)
    wsem = (w0, w1)
    msem = m0
    wid = lax.axis_index("s") * NC + lax.axis_index("c")
    pos_w = seq // NW
    pos_base = wid * pos_w
    nq = pos_w // CH
    n_chunks = batch * nq

    def load_wq(q):
        pltpu.async_copy(wpe_hbm.at[pl.ds(pos_base + q * CH, CH)],
                         wq[q % 2], wsem[q % 2])

    load_wq(0)
    load_wq(1)
    wpb = seq // 256
    mb = wid // wpb
    mcol = (wid % wpb) * 256
    idx_cps = []
    for b in range(batch):
        idx_cps.append(pltpu.async_copy(
            ids_hbm.at[b, pl.ds(pos_base, pos_w)],
            idx_v.at[pl.ds(b * pos_w, pos_w)], osem[b]))
    am_cp = pltpu.async_copy(am_hbm.at[mb, pl.ds(mcol, 256)], am_v, msem)
    for cp in idx_cps:
        cp.wait()

    g_cps = {}
    out_cps = {}

    def gather(i):
        q, b = i // batch, i % batch
        s = i % NBUF
        g_cps[i] = pltpu.async_copy(
            wte_hbm.at[idx_v.at[pl.ds(b * pos_w + q * CH, CH)]],
            bufs[s], gsem[s])

    for i in range(min(NBUF, n_chunks)):
        gather(i)

    am_cp.wait()
    for i in range(256 // 16):
        v = am_v[pl.ds(i * 16, 16)]
        am_v[pl.ds(i * 16, 16)] = (1.0 - v) * (-10000.0)
    am_st = pltpu.async_copy(am_v, mask_hbm.at[0, 0, mb, pl.ds(mcol, 256)],
                             msem)

    for g in range(nq):
        i0 = g * batch
        pltpu.make_async_copy(
            wpe_hbm.at[pl.ds(pos_base + g * CH, CH)],
            wq[g % 2], wsem[g % 2]).wait()
        for b in range(batch):
            g_cps[i0 + b].wait()

        group = tuple(bufs[(i0 + b) % NBUF] for b in range(batch))

        def add_row(r, _, bs=group, w=wq[g % 2]):
            for k in range(D_MODEL // 16):
                sl = pl.ds(k * 16, 16)
                v = w[r, sl]
                for bf in bs:
                    plsc.addupdate(bf.at[r, sl], v)
            return 0
        lax.fori_loop(0, CH, add_row, 0)
        if g + 2 < nq:
            load_wq(g + 2)

        row0 = pos_base + g * CH
        for b in range(batch):
            j = i0 + b
            out_cps[j] = pltpu.async_copy(
                bufs[j % NBUF], out_hbm.at[b, pl.ds(row0, CH)], osem[j % NBUF])
        for j in range(i0 + NBUF, min(i0 + NBUF + batch, n_chunks)):
            out_cps[j - NBUF].wait()
            gather(j)

    for j in range(max(0, n_chunks - NBUF), n_chunks):
        out_cps[j].wait()
    am_st.wait()


@functools.lru_cache(maxsize=None)
def _build(batch, seq):
    mesh = plsc.VectorSubcoreMesh(core_axis_name="c", subcore_axis_name="s")
    pos_w = seq // NW
    return pl.kernel(
        functools.partial(_emb_body, batch, seq),
        out_type=(
            jax.ShapeDtypeStruct((batch, seq, D_MODEL), jnp.float32),
            jax.ShapeDtypeStruct((1, 1, batch, seq), jnp.float32),
        ),
        mesh=mesh,
        scratch_types=[
            pltpu.VMEM((batch * pos_w,), jnp.int32),
            pltpu.VMEM((CH, D_MODEL), jnp.float32),
            pltpu.VMEM((CH, D_MODEL), jnp.float32),
            pltpu.VMEM((CH, D_MODEL), jnp.float32),
            pltpu.VMEM((CH, D_MODEL), jnp.float32),
            pltpu.VMEM((CH, D_MODEL), jnp.float32),
            pltpu.VMEM((CH, D_MODEL), jnp.float32),
            pltpu.VMEM((CH, D_MODEL), jnp.float32),
            pltpu.VMEM((256,), jnp.float32),
        ] + [pltpu.SemaphoreType.DMA] * 13,
    )


def kernel(input_ids, attention_mask, wte, wpe):
    batch, seq = input_ids.shape
    ids = input_ids if input_ids.dtype == jnp.int32 else input_ids.astype(jnp.int32)
    am = (attention_mask if attention_mask.dtype == jnp.float32
          else attention_mask.astype(jnp.float32))
    hidden, ext_mask = _build(batch, seq)(ids, am, wte, wpe)
    return (hidden, ext_mask)
